# baseline (speedup 1.0000x reference)
import functools

import jax
import jax.numpy as jnp
from jax import lax
from jax.experimental import pallas as pl
from jax.experimental.pallas import tpu as pltpu
from jax.experimental.pallas import tpu_sc as plsc

N = 10000
E = 320000
D = 128
NC = 2
NS = 16
NW = NC * NS
EPAD = 327680
EPW = EPAD // NW
CH = 128
NCHUNK = EPW // CH
NBUF = 4
PADROW = N
NACC = 10112
STRIPE = NACC // NS
NDEG = 10240


def _make_sc_agg(with_deg):
    mesh = plsc.VectorSubcoreMesh(
        core_axis_name="c", subcore_axis_name="s", num_cores=NC, num_subcores=NS
    )

    scratch = [
        pltpu.VMEM((NCHUNK, CH), jnp.int32),
        pltpu.VMEM((NCHUNK, CH), jnp.int32),
        pltpu.VMEM((CH, D), jnp.float32),

        pltpu.VMEM_SHARED((NACC, D), jnp.float32),
        pltpu.SemaphoreType.DMA,
        pltpu.SemaphoreType.DMA,
    ]
    if with_deg:
        scratch.append(pltpu.VMEM((NDEG,), jnp.float32))

    if with_deg:
        out_type = [
            jax.ShapeDtypeStruct((NC, NACC, D), jnp.float32),
            jax.ShapeDtypeStruct((NW, NDEG), jnp.float32),
        ]
    else:
        out_type = jax.ShapeDtypeStruct((NC, NACC, D), jnp.float32)

    @functools.partial(
        pl.kernel,
        out_type=out_type,
        mesh=mesh,
        scratch_types=scratch,
        compiler_params=pltpu.CompilerParams(needs_layout_passes=False, use_tc_tiling_on_sc=False),
    )
    def sc_kernel(h_hbm, src_hbm, dst_hbm, zero_hbm, *refs):
        if with_deg:
            (zflat_hbm, out_hbm, deg_hbm, src2d, dst2d, rows,
             acc_sh, semg, sems, deg_l) = refs
        else:
            (out_hbm, src2d, dst2d, rows, acc_sh, semg, sems) = refs
        c = lax.axis_index("c")
        s = lax.axis_index("s")
        wid = s * NC + c

        pltpu.sync_copy(zero_hbm.at[pl.ds(s * STRIPE, STRIPE)],
                        acc_sh.at[pl.ds(s * STRIPE, STRIPE)])
        pltpu.sync_copy(src_hbm.at[wid], src2d)
        pltpu.sync_copy(dst_hbm.at[wid], dst2d)
        if with_deg:
            pltpu.sync_copy(zflat_hbm, deg_l)
            ones = jnp.full((16,), 1.0, jnp.float32)

        plsc.subcore_barrier()

        def body(i, carry):
            cp = pltpu.async_copy(h_hbm.at[src2d.at[i]], rows, semg)
            if with_deg:
                for j in range(CH // 16):
                    d16 = dst2d[i, pl.ds(j * 16, 16)]
                    plsc.addupdate_scatter(deg_l, [d16], ones)
            cp.wait()
            pltpu.async_copy(rows, acc_sh.at[dst2d.at[i]], sems,
                             add=True).wait()
            return carry

        lax.fori_loop(0, NCHUNK, body, 0)

        if with_deg:
            pltpu.sync_copy(deg_l, deg_hbm.at[wid])

        plsc.subcore_barrier()

        pltpu.sync_copy(acc_sh.at[pl.ds(s * STRIPE, STRIPE)],
                        out_hbm.at[c, pl.ds(s * STRIPE, STRIPE)])

    return sc_kernel


_sc_agg_deg = _make_sc_agg(True)
_sc_agg = _make_sc_agg(False)


ROWS_BLK = 1000


def _tc1_body(x_ref, a0_ref, a1_ref, deg_ref, w_ref, b_ref, h_ref):
    agg = a0_ref[0] + a1_ref[0]
    deg = jnp.maximum(deg_ref[...], 1.0)
    h = jnp.dot(x_ref[...], w_ref[:D, :], preferred_element_type=jnp.float32)
    h = h + jnp.dot(agg / deg, w_ref[D:, :], preferred_element_type=jnp.float32)
    h = h + b_ref[...]
    h = jnp.maximum(h, 0.0)
    nrm = jnp.sqrt(jnp.sum(h * h, axis=1, keepdims=True))
    h_ref[...] = h / (nrm + 1e-12)


def _tc2_body(h1_ref, a0_ref, a1_ref, deg_ref, w2_ref, b2_ref, wc_ref, bc_ref,
              out_ref):
    deg = jnp.maximum(deg_ref[...], 1.0)
    agg = (a0_ref[0] + a1_ref[0]) / deg
    h2 = jnp.dot(h1_ref[...], w2_ref[:D, :], preferred_element_type=jnp.float32)
    h2 = h2 + jnp.dot(agg, w2_ref[D:, :], preferred_element_type=jnp.float32)
    h2 = h2 + b2_ref[...]
    out_ref[...] = (jnp.dot(h2, wc_ref[...], preferred_element_type=jnp.float32)
                    + bc_ref[...])


def _row_spec(width):
    return pl.BlockSpec((ROWS_BLK, width), lambda i: (i, 0))


def _acc_spec(c):
    return pl.BlockSpec((1, ROWS_BLK, D), lambda i, c=c: (c, i, 0))


def _full_spec(shape):
    return pl.BlockSpec(shape, lambda i: tuple(0 for _ in shape))


def kernel(x, edge_index, W1, b1, W2, b2, Wc, bc):
    src = edge_index[0].astype(jnp.int32)
    dst = edge_index[1].astype(jnp.int32)

    npad = EPAD - E
    src_r = jnp.concatenate([src, jnp.zeros((npad,), jnp.int32)]).reshape(
        NW, NCHUNK, CH)
    dst_r = jnp.concatenate([dst, jnp.full((npad,), PADROW, jnp.int32)]
                            ).reshape(NW, NCHUNK, CH)

    zeros_big = jnp.zeros((NACC, D), jnp.float32)
    zeros_flat = jnp.zeros((NDEG,), jnp.float32)

    acc1, deg32 = _sc_agg_deg(x, src_r, dst_r, zeros_big, zeros_flat)

    degcol = deg32.sum(axis=0)[:N].reshape(N, 1)

    h1 = pl.pallas_call(
        _tc1_body,
        grid=(N // ROWS_BLK,),
        in_specs=[
            _row_spec(D),
            _acc_spec(0),
            _acc_spec(1),
            pl.BlockSpec((ROWS_BLK, 1), lambda i: (i, 0)),
            _full_spec((2 * D, D)),
            _full_spec((1, D)),
        ],
        out_specs=_row_spec(D),
        out_shape=jax.ShapeDtypeStruct((N, D), jnp.float32),
    )(x, acc1, acc1, degcol, W1, b1.reshape(1, D))

    acc2 = _sc_agg(h1, src_r, dst_r, zeros_big)

    logits = pl.pallas_call(
        _tc2_body,
        grid=(N // ROWS_BLK,),
        in_specs=[
            _row_spec(D),
            _acc_spec(0),
            _acc_spec(1),
            pl.BlockSpec((ROWS_BLK, 1), lambda i: (i, 0)),
            _full_spec((2 * D, D)),
            _full_spec((1, D)),
            _full_spec((D, 16)),
            _full_spec((1, 16)),
        ],
        out_specs=pl.BlockSpec((ROWS_BLK, 16), lambda i: (i, 0)),
        out_shape=jax.ShapeDtypeStruct((N, 16), jnp.float32),
    )(h1, acc2, acc2, degcol, W2, b2.reshape(1, D), Wc, bc.reshape(1, 16))

    return logits


# speedup vs baseline: 2.6822x; 2.6822x over previous
import functools

import jax
import jax.numpy as jnp
from jax import lax
from jax.experimental import pallas as pl
from jax.experimental.pallas import tpu as pltpu
from jax.experimental.pallas import tpu_sc as plsc

N = 10000
E = 320000
D = 128
NC = 2
NS = 16
NW = NC * NS
EPAD = 327680
EPW = EPAD // NW
CH = 128
NCHUNK = EPW // CH
NBUF = 4
PADROW = N
NACC = 10112
STRIPE = NACC // NS
NDEG = 10240


def _make_sc_agg(with_deg):
    mesh = plsc.VectorSubcoreMesh(
        core_axis_name="c", subcore_axis_name="s", num_cores=NC, num_subcores=NS
    )

    scratch = [
        pltpu.VMEM((NCHUNK, CH), jnp.int32),
        pltpu.VMEM((NCHUNK, CH), jnp.int32),
        pltpu.VMEM((CH, D), jnp.float32),

        pltpu.VMEM_SHARED((NACC, D), jnp.float32),
        pltpu.SemaphoreType.DMA,
        pltpu.SemaphoreType.DMA,
    ]
    if with_deg:
        scratch.append(pltpu.VMEM((NDEG,), jnp.float32))

    if with_deg:
        out_type = [
            jax.ShapeDtypeStruct((NC, NACC, D), jnp.float32),
            jax.ShapeDtypeStruct((NW, NDEG), jnp.float32),
        ]
    else:
        out_type = jax.ShapeDtypeStruct((NC, NACC, D), jnp.float32)

    @functools.partial(
        pl.kernel,
        out_type=out_type,
        mesh=mesh,
        scratch_types=scratch,
        compiler_params=pltpu.CompilerParams(needs_layout_passes=False, use_tc_tiling_on_sc=False),
    )
    def sc_kernel(h_hbm, src_hbm, dst_hbm, zero_hbm, *refs):
        if with_deg:
            (zflat_hbm, out_hbm, deg_hbm, src2d, dst2d, rows,
             acc_sh, semg, sems, deg_l) = refs
        else:
            (out_hbm, src2d, dst2d, rows, acc_sh, semg, sems) = refs
        c = lax.axis_index("c")
        s = lax.axis_index("s")
        wid = s * NC + c

        pltpu.sync_copy(zero_hbm.at[pl.ds(s * STRIPE, STRIPE)],
                        acc_sh.at[pl.ds(s * STRIPE, STRIPE)])
        pltpu.sync_copy(src_hbm.at[wid], src2d)
        pltpu.sync_copy(dst_hbm.at[wid], dst2d)
        if with_deg:
            pltpu.sync_copy(zflat_hbm, deg_l)
            ones = jnp.full((16,), 1.0, jnp.float32)

        plsc.subcore_barrier()

        def body(i, carry):
            cp = pltpu.async_copy(h_hbm.at[src2d.at[i]], rows, semg)
            if with_deg:
                for j in range(CH // 16):
                    d16 = dst2d[i, pl.ds(j * 16, 16)]
                    plsc.addupdate_scatter(deg_l, [d16], ones)
            cp.wait()
            pltpu.async_copy(rows, acc_sh.at[dst2d.at[i]], sems,
                             add=True).wait()
            return carry

        lax.fori_loop(0, NCHUNK, body, 0)

        if with_deg:
            pltpu.sync_copy(deg_l, deg_hbm.at[wid])

        plsc.subcore_barrier()

        pltpu.sync_copy(acc_sh.at[pl.ds(s * STRIPE, STRIPE)],
                        out_hbm.at[c, pl.ds(s * STRIPE, STRIPE)])

    return sc_kernel


_sc_agg_deg = _make_sc_agg(True)
_sc_agg = _make_sc_agg(False)


ROWS_BLK = 1000


def _tc1_body(x_ref, a0_ref, a1_ref, deg_ref, w_ref, b_ref, h_ref):
    agg = a0_ref[0] + a1_ref[0]
    deg = jnp.maximum(deg_ref[...], 1.0)
    h = jnp.dot(x_ref[...], w_ref[:D, :], preferred_element_type=jnp.float32)
    h = h + jnp.dot(agg / deg, w_ref[D:, :], preferred_element_type=jnp.float32)
    h = h + b_ref[...]
    h = jnp.maximum(h, 0.0)
    nrm = jnp.sqrt(jnp.sum(h * h, axis=1, keepdims=True))
    h_ref[...] = h / (nrm + 1e-12)


def _tc2_body(h1_ref, a0_ref, a1_ref, deg_ref, w2_ref, b2_ref, wc_ref, bc_ref,
              out_ref):
    deg = jnp.maximum(deg_ref[...], 1.0)
    agg = (a0_ref[0] + a1_ref[0]) / deg
    h2 = jnp.dot(h1_ref[...], w2_ref[:D, :], preferred_element_type=jnp.float32)
    h2 = h2 + jnp.dot(agg, w2_ref[D:, :], preferred_element_type=jnp.float32)
    h2 = h2 + b2_ref[...]
    out_ref[...] = (jnp.dot(h2, wc_ref[...], preferred_element_type=jnp.float32)
                    + bc_ref[...])


def _row_spec(width):
    return pl.BlockSpec((ROWS_BLK, width), lambda i: (i, 0))


def _acc_spec(c):
    return pl.BlockSpec((1, ROWS_BLK, D), lambda i, c=c: (c, i, 0))


def _full_spec(shape):
    return pl.BlockSpec(shape, lambda i: tuple(0 for _ in shape))


def kernel(x, edge_index, W1, b1, W2, b2, Wc, bc):
    src = edge_index[0].astype(jnp.int32)
    dst = edge_index[1].astype(jnp.int32)

    npad = EPAD - E
    pad_ids = jnp.arange(npad, dtype=jnp.int32)
    src_r = jnp.concatenate([src, pad_ids % N]).reshape(NW, NCHUNK, CH)
    dst_r = jnp.concatenate([dst, PADROW + pad_ids % (NACC - N)]
                            ).reshape(NW, NCHUNK, CH)

    zeros_big = jnp.zeros((NACC, D), jnp.float32)
    zeros_flat = jnp.zeros((NDEG,), jnp.float32)

    acc1, deg32 = _sc_agg_deg(x, src_r, dst_r, zeros_big, zeros_flat)

    degcol = deg32.sum(axis=0)[:N].reshape(N, 1)

    h1 = pl.pallas_call(
        _tc1_body,
        grid=(N // ROWS_BLK,),
        in_specs=[
            _row_spec(D),
            _acc_spec(0),
            _acc_spec(1),
            pl.BlockSpec((ROWS_BLK, 1), lambda i: (i, 0)),
            _full_spec((2 * D, D)),
            _full_spec((1, D)),
        ],
        out_specs=_row_spec(D),
        out_shape=jax.ShapeDtypeStruct((N, D), jnp.float32),
    )(x, acc1, acc1, degcol, W1, b1.reshape(1, D))

    acc2 = _sc_agg(h1, src_r, dst_r, zeros_big)

    logits = pl.pallas_call(
        _tc2_body,
        grid=(N // ROWS_BLK,),
        in_specs=[
            _row_spec(D),
            _acc_spec(0),
            _acc_spec(1),
            pl.BlockSpec((ROWS_BLK, 1), lambda i: (i, 0)),
            _full_spec((2 * D, D)),
            _full_spec((1, D)),
            _full_spec((D, 16)),
            _full_spec((1, 16)),
        ],
        out_specs=pl.BlockSpec((ROWS_BLK, 16), lambda i: (i, 0)),
        out_shape=jax.ShapeDtypeStruct((N, 16), jnp.float32),
    )(h1, acc2, acc2, degcol, W2, b2.reshape(1, D), Wc, bc.reshape(1, 16))

    return logits


# speedup vs baseline: 3.8424x; 1.4326x over previous
import functools

import jax
import jax.numpy as jnp
from jax import lax
from jax.experimental import pallas as pl
from jax.experimental.pallas import tpu as pltpu
from jax.experimental.pallas import tpu_sc as plsc

N = 10000
E = 320000
D = 128
NC = 2
NS = 16
NW = NC * NS
EPAD = 327680
EPW = EPAD // NW
CH = 128
NCHUNK = EPW // CH
PADROW = N
NACC = 10112
STRIPE = NACC // NS
NDEG = 10240


def _make_sc_agg(with_deg):
    mesh = plsc.VectorSubcoreMesh(
        core_axis_name="c", subcore_axis_name="s", num_cores=NC, num_subcores=NS
    )

    scratch = [
        pltpu.VMEM((CH,), jnp.int32),
        pltpu.VMEM((CH,), jnp.int32),
        pltpu.VMEM((CH,), jnp.int32),
        pltpu.VMEM((CH,), jnp.int32),
        pltpu.VMEM((CH, D), jnp.float32),
        pltpu.VMEM((CH, D), jnp.float32),
        pltpu.VMEM_SHARED((NACC, D), jnp.float32),
        pltpu.SemaphoreType.DMA,
        pltpu.SemaphoreType.DMA,
        pltpu.SemaphoreType.DMA,
        pltpu.SemaphoreType.DMA,
        pltpu.SemaphoreType.DMA,
        pltpu.SemaphoreType.DMA,
    ]
    if with_deg:
        scratch.append(pltpu.VMEM((NDEG,), jnp.float32))

    if with_deg:
        out_type = [
            jax.ShapeDtypeStruct((NC, NACC, D), jnp.float32),
            jax.ShapeDtypeStruct((NW, NDEG), jnp.float32),
        ]
    else:
        out_type = jax.ShapeDtypeStruct((NC, NACC, D), jnp.float32)

    @functools.partial(
        pl.kernel,
        out_type=out_type,
        mesh=mesh,
        scratch_types=scratch,
        compiler_params=pltpu.CompilerParams(needs_layout_passes=False),
    )
    def sc_kernel(h_hbm, src_hbm, dst_hbm, zero_hbm, *refs):
        if with_deg:
            (zflat_hbm, out_hbm, deg_hbm, s0, s1, d0, d1, r0, r1,
             acc_sh, g0, g1, ss0, ss1, ds0, ds1, deg_l) = refs
        else:
            (out_hbm, s0, s1, d0, d1, r0, r1,
             acc_sh, g0, g1, ss0, ss1, ds0, ds1) = refs
        srcv = (s0, s1)
        dstv = (d0, d1)
        rows = (r0, r1)
        semg = (g0, g1)
        semis = (ss0, ss1)
        semid = (ds0, ds1)
        c = lax.axis_index("c")
        s = lax.axis_index("s")
        wid = s * NC + c
        ebase = wid * EPW

        pltpu.sync_copy(zero_hbm.at[pl.ds(s * STRIPE, STRIPE)],
                        acc_sh.at[pl.ds(s * STRIPE, STRIPE)])
        if with_deg:
            pltpu.sync_copy(zflat_hbm, deg_l)
            ones = jnp.full((16,), 1.0, jnp.float32)

        plsc.subcore_barrier()

        for b in range(2):
            pltpu.sync_copy(src_hbm.at[pl.ds(ebase + b * CH, CH)], srcv[b])
            pltpu.sync_copy(dst_hbm.at[pl.ds(ebase + b * CH, CH)], dstv[b])
            pltpu.async_copy(h_hbm.at[srcv[b]], rows[b], semg[b])

        def body(g, carry):
            for b in range(2):
                i = 2 * g + b
                pltpu.make_async_copy(h_hbm.at[srcv[b]], rows[b],
                                      semg[b]).wait()

                @pl.when(i + 2 < NCHUNK)
                def _pf_src(i=i, b=b):
                    pltpu.async_copy(
                        src_hbm.at[pl.ds(ebase + (i + 2) * CH, CH)],
                        srcv[b], semis[b])

                if with_deg:
                    for j in range(CH // 16):
                        d16 = dstv[b][pl.ds(j * 16, 16)]
                        plsc.addupdate_scatter(deg_l, [d16], ones)

                pltpu.sync_copy(rows[b], acc_sh.at[dstv[b]], add=True)

                @pl.when(i + 2 < NCHUNK)
                def _refill(i=i, b=b):
                    pltpu.async_copy(
                        dst_hbm.at[pl.ds(ebase + (i + 2) * CH, CH)],
                        dstv[b], semid[b])
                    pltpu.make_async_copy(
                        src_hbm.at[pl.ds(ebase + (i + 2) * CH, CH)],
                        srcv[b], semis[b]).wait()
                    pltpu.async_copy(h_hbm.at[srcv[b]], rows[b], semg[b])
                    pltpu.make_async_copy(
                        dst_hbm.at[pl.ds(ebase + (i + 2) * CH, CH)],
                        dstv[b], semid[b]).wait()
            return carry

        lax.fori_loop(0, NCHUNK // 2, body, 0)

        if with_deg:
            pltpu.sync_copy(deg_l, deg_hbm.at[wid])

        plsc.subcore_barrier()

        pltpu.sync_copy(acc_sh.at[pl.ds(s * STRIPE, STRIPE)],
                        out_hbm.at[c, pl.ds(s * STRIPE, STRIPE)])

    return sc_kernel


_sc_agg_deg = _make_sc_agg(True)
_sc_agg = _make_sc_agg(False)


ROWS_BLK = 1000


def _tc1_body(x_ref, a0_ref, a1_ref, deg_ref, w_ref, b_ref, h_ref):
    agg = a0_ref[0] + a1_ref[0]
    deg = jnp.maximum(deg_ref[...], 1.0)
    h = jnp.dot(x_ref[...], w_ref[:D, :], preferred_element_type=jnp.float32)
    h = h + jnp.dot(agg / deg, w_ref[D:, :], preferred_element_type=jnp.float32)
    h = h + b_ref[...]
    h = jnp.maximum(h, 0.0)
    nrm = jnp.sqrt(jnp.sum(h * h, axis=1, keepdims=True))
    h_ref[...] = h / (nrm + 1e-12)


def _tc2_body(h1_ref, a0_ref, a1_ref, deg_ref, w2_ref, b2_ref, wc_ref, bc_ref,
              out_ref):
    deg = jnp.maximum(deg_ref[...], 1.0)
    agg = (a0_ref[0] + a1_ref[0]) / deg
    h2 = jnp.dot(h1_ref[...], w2_ref[:D, :], preferred_element_type=jnp.float32)
    h2 = h2 + jnp.dot(agg, w2_ref[D:, :], preferred_element_type=jnp.float32)
    h2 = h2 + b2_ref[...]
    out_ref[...] = (jnp.dot(h2, wc_ref[...], preferred_element_type=jnp.float32)
                    + bc_ref[...])


def _row_spec(width):
    return pl.BlockSpec((ROWS_BLK, width), lambda i: (i, 0))


def _acc_spec(c):
    return pl.BlockSpec((1, ROWS_BLK, D), lambda i, c=c: (c, i, 0))


def _full_spec(shape):
    return pl.BlockSpec(shape, lambda i: tuple(0 for _ in shape))


def kernel(x, edge_index, W1, b1, W2, b2, Wc, bc):
    src = edge_index[0].astype(jnp.int32)
    dst = edge_index[1].astype(jnp.int32)

    npad = EPAD - E
    pad_ids = jnp.arange(npad, dtype=jnp.int32)
    src_r = jnp.concatenate([src, pad_ids % N])
    dst_r = jnp.concatenate([dst, PADROW + pad_ids % (NACC - N)])

    zeros_big = jnp.zeros((NACC, D), jnp.float32)
    zeros_flat = jnp.zeros((NDEG,), jnp.float32)

    acc1, deg32 = _sc_agg_deg(x, src_r, dst_r, zeros_big, zeros_flat)

    degcol = deg32.sum(axis=0)[:N].reshape(N, 1)

    h1 = pl.pallas_call(
        _tc1_body,
        grid=(N // ROWS_BLK,),
        in_specs=[
            _row_spec(D),
            _acc_spec(0),
            _acc_spec(1),
            pl.BlockSpec((ROWS_BLK, 1), lambda i: (i, 0)),
            _full_spec((2 * D, D)),
            _full_spec((1, D)),
        ],
        out_specs=_row_spec(D),
        out_shape=jax.ShapeDtypeStruct((N, D), jnp.float32),
    )(x, acc1, acc1, degcol, W1, b1.reshape(1, D))

    acc2 = _sc_agg(h1, src_r, dst_r, zeros_big)

    logits = pl.pallas_call(
        _tc2_body,
        grid=(N // ROWS_BLK,),
        in_specs=[
            _row_spec(D),
            _acc_spec(0),
            _acc_spec(1),
            pl.BlockSpec((ROWS_BLK, 1), lambda i: (i, 0)),
            _full_spec((2 * D, D)),
            _full_spec((1, D)),
            _full_spec((D, 16)),
            _full_spec((1, 16)),
        ],
        out_specs=pl.BlockSpec((ROWS_BLK, 16), lambda i: (i, 0)),
        out_shape=jax.ShapeDtypeStruct((N, 16), jnp.float32),
    )(h1, acc2, acc2, degcol, W2, b2.reshape(1, D), Wc, bc.reshape(1, 16))

    return logits
